# Initial kernel scaffold; baseline (speedup 1.0000x reference)
import functools

import jax
import jax.numpy as jnp
from jax import lax
from jax.experimental import pallas as pl
from jax.experimental.pallas import tpu as pltpu
from jax.experimental.pallas import tpu_sc as plsc

NC = 2
NS = 16
NW = NC * NS
CH = 128
LANES = 16
BN = 1000


def _edge_lin_body(ea, we1, be1, we2, be2, ef1, ef2):
    a = ea[...]
    ef1[...] = jnp.dot(a, we1[...], preferred_element_type=jnp.float32) + be1[...]
    ef2[...] = jnp.dot(a, we2[...], preferred_element_type=jnp.float32) + be2[...]


def _edge_lin(ea_pad, We1, be1, We2, be2, BE):
    EP, DE = ea_pad.shape
    DF = We1.shape[1]
    H = We2.shape[1]
    return pl.pallas_call(
        _edge_lin_body,
        grid=(EP // BE,),
        in_specs=[
            pl.BlockSpec((BE, DE), lambda i: (i, 0)),
            pl.BlockSpec((DE, DF), lambda i: (0, 0)),
            pl.BlockSpec((1, DF), lambda i: (0, 0)),
            pl.BlockSpec((DE, H), lambda i: (0, 0)),
            pl.BlockSpec((1, H), lambda i: (0, 0)),
        ],
        out_specs=[
            pl.BlockSpec((BE, DF), lambda i: (i, 0)),
            pl.BlockSpec((BE, H), lambda i: (i, 0)),
        ],
        out_shape=[
            jax.ShapeDtypeStruct((EP, DF), jnp.float32),
            jax.ShapeDtypeStruct((EP, H), jnp.float32),
        ],
    )(ea_pad, We1, be1.reshape(1, -1), We2, be2.reshape(1, -1))


def _sc_conv(x, srcw, dstw, efw, zrows, NP):
    D = x.shape[1]
    CPW = srcw.shape[1]
    stripe = NP // NS
    mesh = plsc.VectorSubcoreMesh(core_axis_name="c", subcore_axis_name="s")

    @functools.partial(
        pl.kernel,
        out_type=jax.ShapeDtypeStruct((NC, NP, D), jnp.float32),
        mesh=mesh,
        scratch_types=[
            pltpu.VMEM((CH,), jnp.int32),
            pltpu.VMEM((CH,), jnp.int32),
            pltpu.VMEM((CH, D), jnp.float32),
            pltpu.VMEM((CH, D), jnp.float32),
            pltpu.VMEM_SHARED((NP, D), jnp.float32),
            pltpu.SemaphoreType.DMA,
            pltpu.SemaphoreType.DMA,
        ],
    )
    def conv(x_hbm, src_hbm, dst_hbm, ef_hbm, z_hbm, out_hbm,
             src_v, dst_v, rows_v, ef_v, acc, sem1, sem2):
        c = lax.axis_index("c")
        s = lax.axis_index("s")
        wid = c * NS + s
        zvec = jnp.zeros((LANES,), jnp.float32)

        pltpu.sync_copy(z_hbm, acc.at[pl.ds(s * stripe, stripe)])
        plsc.subcore_barrier()

        @pl.loop(0, CPW)
        def _chunk(j):
            pltpu.sync_copy(src_hbm.at[wid, j], src_v)
            pltpu.sync_copy(dst_hbm.at[wid, j], dst_v)
            cp_rows = pltpu.async_copy(x_hbm.at[src_v], rows_v, sem1)
            cp_ef = pltpu.async_copy(ef_hbm.at[wid, j], ef_v, sem2)
            cp_rows.wait()
            cp_ef.wait()

            @pl.loop(0, CH)
            def _edge(r):
                for k in range(D // LANES):
                    sl = pl.ds(k * LANES, LANES)
                    rows_v[r, sl] = jnp.maximum(rows_v[r, sl] + ef_v[r, sl], zvec)

            pltpu.sync_copy(rows_v, acc.at[dst_v], add=True)

        plsc.subcore_barrier()
        pltpu.sync_copy(acc.at[pl.ds(s * stripe, stripe)],
                        out_hbm.at[c, pl.ds(s * stripe, stripe)])

    return conv(x, srcw, dstw, efw, zrows)


def _relu(v):
    return jnp.maximum(v, 0.0)


def _mlp1_body(x, a0, a1, wa, ba, wb, bb, out):
    m = x[...] + a0[...] + a1[...]
    t = _relu(jnp.dot(m, wa[...], preferred_element_type=jnp.float32) + ba[...])
    out[...] = _relu(jnp.dot(t, wb[...], preferred_element_type=jnp.float32) + bb[...])


def _node_mlp1(x, a0, a1, W1a, b1a, W1b, b1b):
    n_nodes, DF = x.shape
    H = W1a.shape[1]
    return pl.pallas_call(
        _mlp1_body,
        grid=(n_nodes // BN,),
        in_specs=[
            pl.BlockSpec((BN, DF), lambda i: (i, 0)),
            pl.BlockSpec((BN, DF), lambda i: (i, 0)),
            pl.BlockSpec((BN, DF), lambda i: (i, 0)),
            pl.BlockSpec((DF, H), lambda i: (0, 0)),
            pl.BlockSpec((1, H), lambda i: (0, 0)),
            pl.BlockSpec((H, H), lambda i: (0, 0)),
            pl.BlockSpec((1, H), lambda i: (0, 0)),
        ],
        out_specs=pl.BlockSpec((BN, H), lambda i: (i, 0)),
        out_shape=jax.ShapeDtypeStruct((n_nodes, H), jnp.float32),
    )(x, a0, a1, W1a, b1a.reshape(1, -1), W1b, b1b.reshape(1, -1))


def _mlp2_body(G, h, a0, a1, w2a, b2a, w2b, b2b, wr1, br1, wr2, br2, batch,
               pi_out, pooled_out, acc):
    m = h[...] + a0[...] + a1[...]
    t = _relu(jnp.dot(m, w2a[...], preferred_element_type=jnp.float32) + b2a[...])
    t = _relu(jnp.dot(t, w2b[...], preferred_element_type=jnp.float32) + b2b[...])
    r = _relu(jnp.dot(t, wr1[...], preferred_element_type=jnp.float32) + br1[...])
    z = jnp.dot(r, wr2[...], preferred_element_type=jnp.float32) + br2[...]
    pi = jax.nn.sigmoid(z[:, 0])
    pi_out[0, 0, :] = pi
    b = batch[0, 0, :]
    onehot = (b[:, None] == lax.broadcasted_iota(jnp.int32, (b.shape[0], G), 1))
    contrib = jnp.sum(jnp.where(onehot, pi[:, None], 0.0), axis=0)

    @pl.when(pl.program_id(0) == 0)
    def _():
        acc[...] = jnp.zeros_like(acc)

    acc[...] += contrib[None, :]
    pooled_out[...] = acc[...]


def _node_mlp2(h, a0, a1, W2a, b2a, W2b, b2b, Wr1, br1, Wr2, br2, batch3, G):
    n_nodes, H = h.shape
    HR = Wr1.shape[1]
    nb = n_nodes // BN
    return pl.pallas_call(
        functools.partial(_mlp2_body, G),
        grid=(nb,),
        in_specs=[
            pl.BlockSpec((BN, H), lambda i: (i, 0)),
            pl.BlockSpec((BN, H), lambda i: (i, 0)),
            pl.BlockSpec((BN, H), lambda i: (i, 0)),
            pl.BlockSpec((H, H), lambda i: (0, 0)),
            pl.BlockSpec((1, H), lambda i: (0, 0)),
            pl.BlockSpec((H, H), lambda i: (0, 0)),
            pl.BlockSpec((1, H), lambda i: (0, 0)),
            pl.BlockSpec((H, HR), lambda i: (0, 0)),
            pl.BlockSpec((1, HR), lambda i: (0, 0)),
            pl.BlockSpec((HR, 1), lambda i: (0, 0)),
            pl.BlockSpec((1, 1), lambda i: (0, 0)),
            pl.BlockSpec((1, 1, BN), lambda i: (i, 0, 0)),
        ],
        out_specs=[
            pl.BlockSpec((1, 1, BN), lambda i: (i, 0, 0)),
            pl.BlockSpec((1, G), lambda i: (0, 0)),
        ],
        out_shape=[
            jax.ShapeDtypeStruct((nb, 1, BN), jnp.float32),
            jax.ShapeDtypeStruct((1, G), jnp.float32),
        ],
        scratch_shapes=[pltpu.VMEM((1, G), jnp.float32)],
    )(h, a0, a1, W2a, b2a.reshape(1, -1), W2b, b2b.reshape(1, -1),
      Wr1, br1.reshape(1, -1), Wr2, br2.reshape(1, -1), batch3)


def _final_body(G, pi, batch, btot, pooled, out):
    p = pi[0, 0, :]
    b = batch[0, 0, :]
    onehot = (b[:, None] == lax.broadcasted_iota(jnp.int32, (b.shape[0], G), 1))
    B_b = jnp.sum(jnp.where(onehot, btot[...], 0.0), axis=1)
    exp_b = jnp.sum(jnp.where(onehot, pooled[...], 0.0), axis=1)
    ratio = jnp.minimum(B_b / (exp_b + 1e-12), 1.0)
    out[0, 0, :] = p * ratio


def _finalize(pi3, batch3, B_total, pooled, G):
    nb = pi3.shape[0]
    return pl.pallas_call(
        functools.partial(_final_body, G),
        grid=(nb,),
        in_specs=[
            pl.BlockSpec((1, 1, BN), lambda i: (i, 0, 0)),
            pl.BlockSpec((1, 1, BN), lambda i: (i, 0, 0)),
            pl.BlockSpec((1, G), lambda i: (0, 0)),
            pl.BlockSpec((1, G), lambda i: (0, 0)),
        ],
        out_specs=pl.BlockSpec((1, 1, BN), lambda i: (i, 0, 0)),
        out_shape=jax.ShapeDtypeStruct((nb, 1, BN), jnp.float32),
    )(pi3, batch3, B_total.reshape(1, G), pooled)


def kernel(x, edge_index, edge_attr, batch, B_total,
           We1, be1, W1a, b1a, W1b, b1b,
           We2, be2, W2a, b2a, W2b, b2b,
           Wr1, br1, Wr2, br2):
    n_nodes, DF = x.shape
    E = edge_index.shape[1]
    DE = edge_attr.shape[1]
    H = W1a.shape[1]
    G = B_total.shape[0]

    CPW = -(-E // (NW * CH))
    EP = NW * CPW * CH
    NP = ((n_nodes + 1 + NS - 1) // NS) * NS

    pad = EP - E
    src = jnp.concatenate([edge_index[0], jnp.zeros((pad,), jnp.int32)])
    dst = jnp.concatenate([edge_index[1],
                           jnp.full((pad,), n_nodes, jnp.int32)])
    ea_pad = jnp.concatenate(
        [edge_attr, jnp.zeros((pad, DE), jnp.float32)], axis=0)
    srcw = src.reshape(NW, CPW, CH)
    dstw = dst.reshape(NW, CPW, CH)

    ef1, ef2 = _edge_lin(ea_pad, We1, be1, We2, be2, BE=NW * CH)
    ef1w = ef1.reshape(NW, CPW, CH, DF)
    ef2w = ef2.reshape(NW, CPW, CH, H)

    z128 = jnp.zeros((NP // NS, DF), jnp.float32)
    z64 = jnp.zeros((NP // NS, H), jnp.float32)

    agg1 = _sc_conv(x, srcw, dstw, ef1w, z128, NP)
    h1 = _node_mlp1(x, agg1[0, :n_nodes], agg1[1, :n_nodes],
                    W1a, b1a, W1b, b1b)

    agg2 = _sc_conv(h1, srcw, dstw, ef2w, z64, NP)

    batch3 = batch.reshape(-1, 1, BN)
    pi3, pooled = _node_mlp2(h1, agg2[0, :n_nodes], agg2[1, :n_nodes],
                             W2a, b2a, W2b, b2b, Wr1, br1, Wr2, br2,
                             batch3, G)
    out3 = _finalize(pi3, batch3, B_total, pooled, G)
    return out3.reshape(n_nodes)


# speedup vs baseline: 2.7854x; 2.7854x over previous
import functools

import jax
import jax.numpy as jnp
from jax import lax
from jax.experimental import pallas as pl
from jax.experimental.pallas import tpu as pltpu
from jax.experimental.pallas import tpu_sc as plsc

NC = 2
NS = 16
NW = NC * NS
CH = 128
LANES = 16
BN = 1000


def _edge_lin_body(ea, we1, be1, we2, be2, ef1, ef2):
    a = ea[...]
    ef1[...] = jnp.dot(a, we1[...], preferred_element_type=jnp.float32) + be1[...]
    ef2[...] = jnp.dot(a, we2[...], preferred_element_type=jnp.float32) + be2[...]


def _edge_lin(ea_pad, We1, be1, We2, be2, BE):
    EP, DE = ea_pad.shape
    DF = We1.shape[1]
    H = We2.shape[1]
    return pl.pallas_call(
        _edge_lin_body,
        grid=(EP // BE,),
        in_specs=[
            pl.BlockSpec((BE, DE), lambda i: (i, 0)),
            pl.BlockSpec((DE, DF), lambda i: (0, 0)),
            pl.BlockSpec((1, DF), lambda i: (0, 0)),
            pl.BlockSpec((DE, H), lambda i: (0, 0)),
            pl.BlockSpec((1, H), lambda i: (0, 0)),
        ],
        out_specs=[
            pl.BlockSpec((BE, DF), lambda i: (i, 0)),
            pl.BlockSpec((BE, H), lambda i: (i, 0)),
        ],
        out_shape=[
            jax.ShapeDtypeStruct((EP, DF), jnp.float32),
            jax.ShapeDtypeStruct((EP, H), jnp.float32),
        ],
    )(ea_pad, We1, be1.reshape(1, -1), We2, be2.reshape(1, -1))


def _sc_conv(x, srcw, dstw, efw, zrows, NP):
    D = x.shape[1]
    CPW = srcw.shape[1]
    stripe = NP // NS
    mesh = plsc.VectorSubcoreMesh(core_axis_name="c", subcore_axis_name="s",
                                  num_cores=NC, num_subcores=NS)

    @functools.partial(
        pl.kernel,
        out_type=jax.ShapeDtypeStruct((NC, NP, D), jnp.float32),
        mesh=mesh,
        compiler_params=pltpu.CompilerParams(use_tc_tiling_on_sc=False),
        scratch_types=[
            pltpu.VMEM((CH,), jnp.int32),
            pltpu.VMEM((CH,), jnp.int32),
            pltpu.VMEM((CH, D), jnp.float32),
            pltpu.VMEM((CH, D), jnp.float32),
            pltpu.VMEM_SHARED((NP, D), jnp.float32),
            pltpu.SemaphoreType.DMA,
            pltpu.SemaphoreType.DMA,
        ],
    )
    def conv(x_hbm, src_hbm, dst_hbm, ef_hbm, z_hbm, out_hbm,
             src_v, dst_v, rows_v, ef_v, acc, sem1, sem2):
        c = lax.axis_index("c")
        s = lax.axis_index("s")
        wid = c * NS + s
        zvec = jnp.zeros((LANES,), jnp.float32)

        pltpu.sync_copy(z_hbm, acc.at[pl.ds(s * stripe, stripe)])
        plsc.subcore_barrier()

        @pl.loop(0, CPW)
        def _chunk(j):
            pltpu.sync_copy(src_hbm.at[wid, j], src_v)
            pltpu.sync_copy(dst_hbm.at[wid, j], dst_v)
            cp_rows = pltpu.async_copy(x_hbm.at[src_v], rows_v, sem1)
            cp_ef = pltpu.async_copy(ef_hbm.at[wid, j], ef_v, sem2)
            cp_rows.wait()
            cp_ef.wait()

            @pl.loop(0, CH)
            def _edge(r):
                for k in range(D // LANES):
                    sl = pl.ds(k * LANES, LANES)
                    rows_v[r, sl] = jnp.maximum(rows_v[r, sl] + ef_v[r, sl], zvec)

            pltpu.sync_copy(rows_v, acc.at[dst_v], add=True)

        plsc.subcore_barrier()
        pltpu.sync_copy(acc.at[pl.ds(s * stripe, stripe)],
                        out_hbm.at[c, pl.ds(s * stripe, stripe)])

    return conv(x, srcw, dstw, efw, zrows)


def _relu(v):
    return jnp.maximum(v, 0.0)


def _mlp1_body(x, a0, a1, wa, ba, wb, bb, out):
    m = x[...] + a0[...] + a1[...]
    t = _relu(jnp.dot(m, wa[...], preferred_element_type=jnp.float32) + ba[...])
    out[...] = _relu(jnp.dot(t, wb[...], preferred_element_type=jnp.float32) + bb[...])


def _node_mlp1(x, a0, a1, W1a, b1a, W1b, b1b):
    n_nodes, DF = x.shape
    H = W1a.shape[1]
    return pl.pallas_call(
        _mlp1_body,
        grid=(n_nodes // BN,),
        in_specs=[
            pl.BlockSpec((BN, DF), lambda i: (i, 0)),
            pl.BlockSpec((BN, DF), lambda i: (i, 0)),
            pl.BlockSpec((BN, DF), lambda i: (i, 0)),
            pl.BlockSpec((DF, H), lambda i: (0, 0)),
            pl.BlockSpec((1, H), lambda i: (0, 0)),
            pl.BlockSpec((H, H), lambda i: (0, 0)),
            pl.BlockSpec((1, H), lambda i: (0, 0)),
        ],
        out_specs=pl.BlockSpec((BN, H), lambda i: (i, 0)),
        out_shape=jax.ShapeDtypeStruct((n_nodes, H), jnp.float32),
    )(x, a0, a1, W1a, b1a.reshape(1, -1), W1b, b1b.reshape(1, -1))


def _mlp2_body(G, h, a0, a1, w2a, b2a, w2b, b2b, wr1, br1, wr2, br2, batch,
               pi_out, pooled_out, acc):
    m = h[...] + a0[...] + a1[...]
    t = _relu(jnp.dot(m, w2a[...], preferred_element_type=jnp.float32) + b2a[...])
    t = _relu(jnp.dot(t, w2b[...], preferred_element_type=jnp.float32) + b2b[...])
    r = _relu(jnp.dot(t, wr1[...], preferred_element_type=jnp.float32) + br1[...])
    z = jnp.dot(r, wr2[...], preferred_element_type=jnp.float32) + br2[...]
    pi = jax.nn.sigmoid(z[:, 0])
    pi_out[0, 0, :] = pi
    b = batch[0, 0, :]
    onehot = (b[:, None] == lax.broadcasted_iota(jnp.int32, (b.shape[0], G), 1))
    contrib = jnp.sum(jnp.where(onehot, pi[:, None], 0.0), axis=0)

    @pl.when(pl.program_id(0) == 0)
    def _():
        acc[...] = jnp.zeros_like(acc)

    acc[...] += contrib[None, :]
    pooled_out[...] = acc[...]


def _node_mlp2(h, a0, a1, W2a, b2a, W2b, b2b, Wr1, br1, Wr2, br2, batch3, G):
    n_nodes, H = h.shape
    HR = Wr1.shape[1]
    nb = n_nodes // BN
    return pl.pallas_call(
        functools.partial(_mlp2_body, G),
        grid=(nb,),
        in_specs=[
            pl.BlockSpec((BN, H), lambda i: (i, 0)),
            pl.BlockSpec((BN, H), lambda i: (i, 0)),
            pl.BlockSpec((BN, H), lambda i: (i, 0)),
            pl.BlockSpec((H, H), lambda i: (0, 0)),
            pl.BlockSpec((1, H), lambda i: (0, 0)),
            pl.BlockSpec((H, H), lambda i: (0, 0)),
            pl.BlockSpec((1, H), lambda i: (0, 0)),
            pl.BlockSpec((H, HR), lambda i: (0, 0)),
            pl.BlockSpec((1, HR), lambda i: (0, 0)),
            pl.BlockSpec((HR, 1), lambda i: (0, 0)),
            pl.BlockSpec((1, 1), lambda i: (0, 0)),
            pl.BlockSpec((1, 1, BN), lambda i: (i, 0, 0)),
        ],
        out_specs=[
            pl.BlockSpec((1, 1, BN), lambda i: (i, 0, 0)),
            pl.BlockSpec((1, G), lambda i: (0, 0)),
        ],
        out_shape=[
            jax.ShapeDtypeStruct((nb, 1, BN), jnp.float32),
            jax.ShapeDtypeStruct((1, G), jnp.float32),
        ],
        scratch_shapes=[pltpu.VMEM((1, G), jnp.float32)],
    )(h, a0, a1, W2a, b2a.reshape(1, -1), W2b, b2b.reshape(1, -1),
      Wr1, br1.reshape(1, -1), Wr2, br2.reshape(1, -1), batch3)


def _final_body(G, pi, batch, btot, pooled, out):
    p = pi[0, 0, :]
    b = batch[0, 0, :]
    onehot = (b[:, None] == lax.broadcasted_iota(jnp.int32, (b.shape[0], G), 1))
    B_b = jnp.sum(jnp.where(onehot, btot[...], 0.0), axis=1)
    exp_b = jnp.sum(jnp.where(onehot, pooled[...], 0.0), axis=1)
    ratio = jnp.minimum(B_b / (exp_b + 1e-12), 1.0)
    out[0, 0, :] = p * ratio


def _finalize(pi3, batch3, B_total, pooled, G):
    nb = pi3.shape[0]
    return pl.pallas_call(
        functools.partial(_final_body, G),
        grid=(nb,),
        in_specs=[
            pl.BlockSpec((1, 1, BN), lambda i: (i, 0, 0)),
            pl.BlockSpec((1, 1, BN), lambda i: (i, 0, 0)),
            pl.BlockSpec((1, G), lambda i: (0, 0)),
            pl.BlockSpec((1, G), lambda i: (0, 0)),
        ],
        out_specs=pl.BlockSpec((1, 1, BN), lambda i: (i, 0, 0)),
        out_shape=jax.ShapeDtypeStruct((nb, 1, BN), jnp.float32),
    )(pi3, batch3, B_total.reshape(1, G), pooled)


def kernel(x, edge_index, edge_attr, batch, B_total,
           We1, be1, W1a, b1a, W1b, b1b,
           We2, be2, W2a, b2a, W2b, b2b,
           Wr1, br1, Wr2, br2):
    n_nodes, DF = x.shape
    E = edge_index.shape[1]
    DE = edge_attr.shape[1]
    H = W1a.shape[1]
    G = B_total.shape[0]

    CPW = -(-E // (NW * CH))
    EP = NW * CPW * CH
    NP = ((n_nodes + 1 + 127) // 128) * 128

    pad = EP - E
    src = jnp.concatenate([edge_index[0], jnp.zeros((pad,), jnp.int32)])
    dst = jnp.concatenate([edge_index[1],
                           jnp.full((pad,), n_nodes, jnp.int32)])
    ea_pad = jnp.concatenate(
        [edge_attr, jnp.zeros((pad, DE), jnp.float32)], axis=0)
    srcw = src.reshape(NW, CPW, CH)
    dstw = dst.reshape(NW, CPW, CH)

    ef1, ef2 = _edge_lin(ea_pad, We1, be1, We2, be2, BE=NW * CH)
    ef1w = ef1.reshape(NW, CPW, CH, DF)
    ef2w = ef2.reshape(NW, CPW, CH, H)

    z128 = jnp.zeros((NP // NS, DF), jnp.float32)
    z64 = jnp.zeros((NP // NS, H), jnp.float32)

    agg1 = _sc_conv(x, srcw, dstw, ef1w, z128, NP)
    h1 = _node_mlp1(x, agg1[0, :n_nodes], agg1[1, :n_nodes],
                    W1a, b1a, W1b, b1b)

    agg2 = _sc_conv(h1, srcw, dstw, ef2w, z64, NP)

    batch3 = batch.reshape(-1, 1, BN)
    pi3, pooled = _node_mlp2(h1, agg2[0, :n_nodes], agg2[1, :n_nodes],
                             W2a, b2a, W2b, b2b, Wr1, br1, Wr2, br2,
                             batch3, G)
    out3 = _finalize(pi3, batch3, B_total, pooled, G)
    return out3.reshape(n_nodes)


# speedup vs baseline: 2.7943x; 1.0032x over previous
import functools

import jax
import jax.numpy as jnp
from jax import lax
from jax.experimental import pallas as pl
from jax.experimental.pallas import tpu as pltpu
from jax.experimental.pallas import tpu_sc as plsc

NC = 2
NS = 16
NW = NC * NS
CH = 128
LANES = 16
BN = 1000


def _edge_lin_body(ea, we1, be1, we2, be2, ef1, ef2):
    a = ea[...]
    ef1[...] = jnp.dot(a, we1[...], preferred_element_type=jnp.float32) + be1[...]
    ef2[...] = jnp.dot(a, we2[...], preferred_element_type=jnp.float32) + be2[...]


def _edge_lin(ea_pad, We1, be1, We2, be2, BE):
    EP, DE = ea_pad.shape
    DF = We1.shape[1]
    H = We2.shape[1]
    return pl.pallas_call(
        _edge_lin_body,
        grid=(EP // BE,),
        in_specs=[
            pl.BlockSpec((BE, DE), lambda i: (i, 0)),
            pl.BlockSpec((DE, DF), lambda i: (0, 0)),
            pl.BlockSpec((1, DF), lambda i: (0, 0)),
            pl.BlockSpec((DE, H), lambda i: (0, 0)),
            pl.BlockSpec((1, H), lambda i: (0, 0)),
        ],
        out_specs=[
            pl.BlockSpec((BE, DF), lambda i: (i, 0)),
            pl.BlockSpec((BE, H), lambda i: (i, 0)),
        ],
        out_shape=[
            jax.ShapeDtypeStruct((EP, DF), jnp.float32),
            jax.ShapeDtypeStruct((EP, H), jnp.float32),
        ],
    )(ea_pad, We1, be1.reshape(1, -1), We2, be2.reshape(1, -1))


def _sc_conv(x, srcw, dstw, efw, zrows, NP):
    D = x.shape[1]
    CPW, CHB = srcw.shape[1], srcw.shape[2]
    stripe = NP // NS
    mesh = plsc.VectorSubcoreMesh(core_axis_name="c", subcore_axis_name="s",
                                  num_cores=NC, num_subcores=NS)

    @functools.partial(
        pl.kernel,
        out_type=jax.ShapeDtypeStruct((NC, NP, D), jnp.float32),
        mesh=mesh,
        compiler_params=pltpu.CompilerParams(use_tc_tiling_on_sc=False),
        scratch_types=[
            pltpu.VMEM((CPW, CHB), jnp.int32),
            pltpu.VMEM((CHB,), jnp.int32),
            pltpu.VMEM((CHB,), jnp.int32),
            pltpu.VMEM((CHB, D), jnp.float32),
            pltpu.VMEM((CHB, D), jnp.float32),
            pltpu.VMEM((CHB, D), jnp.float32),
            pltpu.VMEM((CHB, D), jnp.float32),
            pltpu.VMEM_SHARED((NP, D), jnp.float32),
            pltpu.SemaphoreType.DMA,
            pltpu.SemaphoreType.DMA,
            pltpu.SemaphoreType.DMA,
            pltpu.SemaphoreType.DMA,
            pltpu.SemaphoreType.DMA,
            pltpu.SemaphoreType.DMA,
        ],
    )
    def conv(x_hbm, src_hbm, dst_hbm, ef_hbm, z_hbm, out_hbm,
             src_all, dstb0, dstb1, rows0, rows1, ef0, ef1, acc,
             semr0, semr1, seme0, seme1, semd0, semd1):
        c = lax.axis_index("c")
        s = lax.axis_index("s")
        wid = c * NS + s
        zvec = jnp.zeros((LANES,), jnp.float32)
        rows = (rows0, rows1)
        efs = (ef0, ef1)
        dsts = (dstb0, dstb1)
        semr = (semr0, semr1)
        seme = (seme0, seme1)
        semd = (semd0, semd1)

        pltpu.sync_copy(src_hbm.at[wid], src_all)
        pltpu.sync_copy(z_hbm, acc.at[pl.ds(s * stripe, stripe)])
        plsc.subcore_barrier()

        def issue(j, b):
            pltpu.async_copy(x_hbm.at[src_all.at[j]], rows[b], semr[b])
            pltpu.async_copy(ef_hbm.at[wid, j], efs[b], seme[b])
            pltpu.async_copy(dst_hbm.at[wid, j], dsts[b], semd[b])

        def wait(j, b):
            pltpu.make_async_copy(x_hbm.at[src_all.at[j]], rows[b], semr[b]).wait()
            pltpu.make_async_copy(ef_hbm.at[wid, j], efs[b], seme[b]).wait()
            pltpu.make_async_copy(dst_hbm.at[wid, j], dsts[b], semd[b]).wait()

        def process(b):
            rv, ev = rows[b], efs[b]

            @pl.loop(0, CHB, unroll=4)
            def _edge(r):
                for k in range(D // LANES):
                    sl = pl.ds(k * LANES, LANES)
                    rv[r, sl] = jnp.maximum(rv[r, sl] + ev[r, sl], zvec)

            pltpu.sync_copy(rv, acc.at[dsts[b]], add=True)

        issue(0, 0)

        @pl.loop(0, CPW // 2)
        def _pair(p):
            j0 = 2 * p
            issue(j0 + 1, 1)
            wait(j0, 0)
            process(0)

            @pl.when(p + 1 < CPW // 2)
            def _():
                issue(j0 + 2, 0)

            wait(j0 + 1, 1)
            process(1)

        plsc.subcore_barrier()
        pltpu.sync_copy(acc.at[pl.ds(s * stripe, stripe)],
                        out_hbm.at[c, pl.ds(s * stripe, stripe)])

    return conv(x, srcw, dstw, efw, zrows)


def _relu(v):
    return jnp.maximum(v, 0.0)


def _mlp1_body(x, a0, a1, wa, ba, wb, bb, out):
    m = x[...] + a0[...] + a1[...]
    t = _relu(jnp.dot(m, wa[...], preferred_element_type=jnp.float32) + ba[...])
    out[...] = _relu(jnp.dot(t, wb[...], preferred_element_type=jnp.float32) + bb[...])


def _node_mlp1(x, a0, a1, W1a, b1a, W1b, b1b):
    n_nodes, DF = x.shape
    H = W1a.shape[1]
    return pl.pallas_call(
        _mlp1_body,
        grid=(n_nodes // BN,),
        in_specs=[
            pl.BlockSpec((BN, DF), lambda i: (i, 0)),
            pl.BlockSpec((BN, DF), lambda i: (i, 0)),
            pl.BlockSpec((BN, DF), lambda i: (i, 0)),
            pl.BlockSpec((DF, H), lambda i: (0, 0)),
            pl.BlockSpec((1, H), lambda i: (0, 0)),
            pl.BlockSpec((H, H), lambda i: (0, 0)),
            pl.BlockSpec((1, H), lambda i: (0, 0)),
        ],
        out_specs=pl.BlockSpec((BN, H), lambda i: (i, 0)),
        out_shape=jax.ShapeDtypeStruct((n_nodes, H), jnp.float32),
    )(x, a0, a1, W1a, b1a.reshape(1, -1), W1b, b1b.reshape(1, -1))


def _mlp2_body(G, h, a0, a1, w2a, b2a, w2b, b2b, wr1, br1, wr2, br2, batch,
               pi_out, pooled_out, acc):
    m = h[...] + a0[...] + a1[...]
    t = _relu(jnp.dot(m, w2a[...], preferred_element_type=jnp.float32) + b2a[...])
    t = _relu(jnp.dot(t, w2b[...], preferred_element_type=jnp.float32) + b2b[...])
    r = _relu(jnp.dot(t, wr1[...], preferred_element_type=jnp.float32) + br1[...])
    z = jnp.dot(r, wr2[...], preferred_element_type=jnp.float32) + br2[...]
    pi = jax.nn.sigmoid(z[:, 0])
    pi_out[0, 0, :] = pi
    b = batch[0, 0, :]
    onehot = (b[:, None] == lax.broadcasted_iota(jnp.int32, (b.shape[0], G), 1))
    contrib = jnp.sum(jnp.where(onehot, pi[:, None], 0.0), axis=0)

    @pl.when(pl.program_id(0) == 0)
    def _():
        acc[...] = jnp.zeros_like(acc)

    acc[...] += contrib[None, :]
    pooled_out[...] = acc[...]


def _node_mlp2(h, a0, a1, W2a, b2a, W2b, b2b, Wr1, br1, Wr2, br2, batch3, G):
    n_nodes, H = h.shape
    HR = Wr1.shape[1]
    nb = n_nodes // BN
    return pl.pallas_call(
        functools.partial(_mlp2_body, G),
        grid=(nb,),
        in_specs=[
            pl.BlockSpec((BN, H), lambda i: (i, 0)),
            pl.BlockSpec((BN, H), lambda i: (i, 0)),
            pl.BlockSpec((BN, H), lambda i: (i, 0)),
            pl.BlockSpec((H, H), lambda i: (0, 0)),
            pl.BlockSpec((1, H), lambda i: (0, 0)),
            pl.BlockSpec((H, H), lambda i: (0, 0)),
            pl.BlockSpec((1, H), lambda i: (0, 0)),
            pl.BlockSpec((H, HR), lambda i: (0, 0)),
            pl.BlockSpec((1, HR), lambda i: (0, 0)),
            pl.BlockSpec((HR, 1), lambda i: (0, 0)),
            pl.BlockSpec((1, 1), lambda i: (0, 0)),
            pl.BlockSpec((1, 1, BN), lambda i: (i, 0, 0)),
        ],
        out_specs=[
            pl.BlockSpec((1, 1, BN), lambda i: (i, 0, 0)),
            pl.BlockSpec((1, G), lambda i: (0, 0)),
        ],
        out_shape=[
            jax.ShapeDtypeStruct((nb, 1, BN), jnp.float32),
            jax.ShapeDtypeStruct((1, G), jnp.float32),
        ],
        scratch_shapes=[pltpu.VMEM((1, G), jnp.float32)],
    )(h, a0, a1, W2a, b2a.reshape(1, -1), W2b, b2b.reshape(1, -1),
      Wr1, br1.reshape(1, -1), Wr2, br2.reshape(1, -1), batch3)


def _final_body(G, pi, batch, btot, pooled, out):
    p = pi[0, 0, :]
    b = batch[0, 0, :]
    onehot = (b[:, None] == lax.broadcasted_iota(jnp.int32, (b.shape[0], G), 1))
    B_b = jnp.sum(jnp.where(onehot, btot[...], 0.0), axis=1)
    exp_b = jnp.sum(jnp.where(onehot, pooled[...], 0.0), axis=1)
    ratio = jnp.minimum(B_b / (exp_b + 1e-12), 1.0)
    out[0, 0, :] = p * ratio


def _finalize(pi3, batch3, B_total, pooled, G):
    nb = pi3.shape[0]
    return pl.pallas_call(
        functools.partial(_final_body, G),
        grid=(nb,),
        in_specs=[
            pl.BlockSpec((1, 1, BN), lambda i: (i, 0, 0)),
            pl.BlockSpec((1, 1, BN), lambda i: (i, 0, 0)),
            pl.BlockSpec((1, G), lambda i: (0, 0)),
            pl.BlockSpec((1, G), lambda i: (0, 0)),
        ],
        out_specs=pl.BlockSpec((1, 1, BN), lambda i: (i, 0, 0)),
        out_shape=jax.ShapeDtypeStruct((nb, 1, BN), jnp.float32),
    )(pi3, batch3, B_total.reshape(1, G), pooled)


def kernel(x, edge_index, edge_attr, batch, B_total,
           We1, be1, W1a, b1a, W1b, b1b,
           We2, be2, W2a, b2a, W2b, b2b,
           Wr1, br1, Wr2, br2):
    n_nodes, DF = x.shape
    E = edge_index.shape[1]
    DE = edge_attr.shape[1]
    H = W1a.shape[1]
    G = B_total.shape[0]

    CPW = -(-E // (NW * CH))
    CPW += CPW % 2
    EP = NW * CPW * CH
    NP = ((n_nodes + 1 + 127) // 128) * 128

    pad = EP - E
    src = jnp.concatenate([edge_index[0], jnp.zeros((pad,), jnp.int32)])
    dst = jnp.concatenate([edge_index[1],
                           jnp.full((pad,), n_nodes, jnp.int32)])
    ea_pad = jnp.concatenate(
        [edge_attr, jnp.zeros((pad, DE), jnp.float32)], axis=0)
    CH1, CH2 = 64, 128
    CPW1, CPW2 = EP // (NW * CH1), EP // (NW * CH2)
    srcw1 = src.reshape(NW, CPW1, CH1)
    dstw1 = dst.reshape(NW, CPW1, CH1)
    srcw2 = src.reshape(NW, CPW2, CH2)
    dstw2 = dst.reshape(NW, CPW2, CH2)

    ef1, ef2 = _edge_lin(ea_pad, We1, be1, We2, be2, BE=NW * CH)
    ef1w = ef1.reshape(NW, CPW1, CH1, DF)
    ef2w = ef2.reshape(NW, CPW2, CH2, H)

    z128 = jnp.zeros((NP // NS, DF), jnp.float32)
    z64 = jnp.zeros((NP // NS, H), jnp.float32)

    agg1 = _sc_conv(x, srcw1, dstw1, ef1w, z128, NP)
    h1 = _node_mlp1(x, agg1[0, :n_nodes], agg1[1, :n_nodes],
                    W1a, b1a, W1b, b1b)

    agg2 = _sc_conv(h1, srcw2, dstw2, ef2w, z64, NP)

    batch3 = batch.reshape(-1, 1, BN)
    pi3, pooled = _node_mlp2(h1, agg2[0, :n_nodes], agg2[1, :n_nodes],
                             W2a, b2a, W2b, b2b, Wr1, br1, Wr2, br2,
                             batch3, G)
    out3 = _finalize(pi3, batch3, B_total, pooled, G)
    return out3.reshape(n_nodes)


# speedup vs baseline: 3.0268x; 1.0832x over previous
import functools

import jax
import jax.numpy as jnp
from jax import lax
from jax.experimental import pallas as pl
from jax.experimental.pallas import tpu as pltpu
from jax.experimental.pallas import tpu_sc as plsc

NC = 2
NS = 16
NW = NC * NS
LANES = 16
BN = 1000
MASK_HI = jnp.int32(-65536)


def _pack_bf16_pair(lo, hi):
    bl = lax.bitcast_convert_type(lo, jnp.uint32)
    bh = lax.bitcast_convert_type(hi, jnp.uint32)
    bl = (bl + 0x7FFF + ((bl >> 16) & 1)) >> 16
    bh = (bh + 0x7FFF + ((bh >> 16) & 1)) >> 16
    return lax.bitcast_convert_type(bl | (bh << 16), jnp.int32)


def _edge_lin_body(ea, we, be, efp):
    ef = jnp.dot(ea[...], we[...], preferred_element_type=jnp.float32) + be[...]
    d2 = ef.shape[1] // 2
    efp[...] = _pack_bf16_pair(ef[:, :d2], ef[:, d2:])


def _edge_lin(ea_pad, We, be, BE):
    EP, DE = ea_pad.shape
    D = We.shape[1]
    return pl.pallas_call(
        _edge_lin_body,
        grid=(EP // BE,),
        in_specs=[
            pl.BlockSpec((BE, DE), lambda i: (i, 0)),
            pl.BlockSpec((DE, D), lambda i: (0, 0)),
            pl.BlockSpec((1, D), lambda i: (0, 0)),
        ],
        out_specs=pl.BlockSpec((BE, D // 2), lambda i: (i, 0)),
        out_shape=jax.ShapeDtypeStruct((EP, D // 2), jnp.int32),
    )(ea_pad, We, be.reshape(1, -1))


def _pack_rows_body(x, xp):
    d2 = x.shape[1] // 2
    xp[...] = _pack_bf16_pair(x[..., :d2], x[..., d2:])


def _pack_rows(x):
    n, D = x.shape
    return pl.pallas_call(
        _pack_rows_body,
        grid=(n // BN,),
        in_specs=[pl.BlockSpec((BN, D), lambda i: (i, 0))],
        out_specs=pl.BlockSpec((BN, D // 2), lambda i: (i, 0)),
        out_shape=jax.ShapeDtypeStruct((n, D // 2), jnp.int32),
    )(x)


def _sc_conv(xp, srcw, dstw, efw, zrows, NP, D):
    D2 = D // 2
    CPW, CHB = srcw.shape[1], srcw.shape[2]
    stripe = NP // NS
    mesh = plsc.VectorSubcoreMesh(core_axis_name="c", subcore_axis_name="s",
                                  num_cores=NC, num_subcores=NS)

    @functools.partial(
        pl.kernel,
        out_type=jax.ShapeDtypeStruct((NC, NP, D), jnp.float32),
        mesh=mesh,
        compiler_params=pltpu.CompilerParams(use_tc_tiling_on_sc=False,
                                             needs_layout_passes=False),
        scratch_types=[
            pltpu.VMEM((CPW, CHB), jnp.int32),
            pltpu.VMEM((CPW, CHB), jnp.int32),
            pltpu.VMEM((CHB, D2), jnp.int32),
            pltpu.VMEM((CHB, D2), jnp.int32),
            pltpu.VMEM((CHB, D2), jnp.int32),
            pltpu.VMEM((CHB, D2), jnp.int32),
            pltpu.VMEM((CHB, D), jnp.float32),
            pltpu.VMEM_SHARED((NP, D), jnp.float32),
            pltpu.SemaphoreType.DMA,
            pltpu.SemaphoreType.DMA,
            pltpu.SemaphoreType.DMA,
            pltpu.SemaphoreType.DMA,
        ],
    )
    def conv(x_hbm, src_hbm, dst_hbm, ef_hbm, z_hbm, out_hbm,
             src_all, dst_all, rows0, rows1, ef0, ef1, msg, acc,
             semr0, semr1, seme0, seme1):
        c = lax.axis_index("c")
        s = lax.axis_index("s")
        wid = c * NS + s
        zvec = jnp.zeros((LANES,), jnp.float32)
        rows = (rows0, rows1)
        efs = (ef0, ef1)
        semr = (semr0, semr1)
        seme = (seme0, seme1)

        pltpu.sync_copy(src_hbm.at[wid], src_all)
        pltpu.sync_copy(dst_hbm.at[wid], dst_all)
        pltpu.sync_copy(z_hbm, acc.at[pl.ds(s * stripe, stripe)])
        plsc.subcore_barrier()

        def issue(j, b):
            pltpu.async_copy(x_hbm.at[src_all.at[j]], rows[b], semr[b])
            pltpu.async_copy(ef_hbm.at[wid, j], efs[b], seme[b])

        def wait(j, b):
            pltpu.make_async_copy(x_hbm.at[src_all.at[j]], rows[b], semr[b]).wait()
            pltpu.make_async_copy(ef_hbm.at[wid, j], efs[b], seme[b]).wait()

        def process(j, b):
            rv, ev = rows[b], efs[b]

            @pl.loop(0, CHB, unroll=4)
            def _edge(r):
                for k in range(D2 // LANES):
                    sl = pl.ds(k * LANES, LANES)
                    vr = rv[r, sl]
                    ve = ev[r, sl]
                    lo = (plsc.bitcast(vr << 16, jnp.float32)
                          + plsc.bitcast(ve << 16, jnp.float32))
                    hi = (plsc.bitcast(vr & MASK_HI, jnp.float32)
                          + plsc.bitcast(ve & MASK_HI, jnp.float32))
                    msg[r, sl] = jnp.maximum(lo, zvec)
                    msg[r, pl.ds(D2 + k * LANES, LANES)] = jnp.maximum(hi, zvec)

            pltpu.sync_copy(msg, acc.at[dst_all.at[j]], add=True)

        issue(0, 0)

        @pl.loop(0, CPW // 2)
        def _pair(p):
            j0 = 2 * p
            issue(j0 + 1, 1)
            wait(j0, 0)
            process(j0, 0)

            @pl.when(p + 1 < CPW // 2)
            def _():
                issue(j0 + 2, 0)

            wait(j0 + 1, 1)
            process(j0 + 1, 1)

        plsc.subcore_barrier()
        pltpu.sync_copy(acc.at[pl.ds(s * stripe, stripe)],
                        out_hbm.at[c, pl.ds(s * stripe, stripe)])

    return conv(xp, srcw, dstw, efw, zrows)


def _relu(v):
    return jnp.maximum(v, 0.0)


def _mlp1_body(x, a0, a1, wa, ba, wb, bb, out, outp):
    m = x[...] + a0[...] + a1[...]
    t = _relu(jnp.dot(m, wa[...], preferred_element_type=jnp.float32) + ba[...])
    h = _relu(jnp.dot(t, wb[...], preferred_element_type=jnp.float32) + bb[...])
    out[...] = h
    d2 = h.shape[1] // 2
    outp[...] = _pack_bf16_pair(h[:, :d2], h[:, d2:])


def _node_mlp1(x, a0, a1, W1a, b1a, W1b, b1b):
    n_nodes, DF = x.shape
    H = W1a.shape[1]
    return pl.pallas_call(
        _mlp1_body,
        grid=(n_nodes // BN,),
        in_specs=[
            pl.BlockSpec((BN, DF), lambda i: (i, 0)),
            pl.BlockSpec((BN, DF), lambda i: (i, 0)),
            pl.BlockSpec((BN, DF), lambda i: (i, 0)),
            pl.BlockSpec((DF, H), lambda i: (0, 0)),
            pl.BlockSpec((1, H), lambda i: (0, 0)),
            pl.BlockSpec((H, H), lambda i: (0, 0)),
            pl.BlockSpec((1, H), lambda i: (0, 0)),
        ],
        out_specs=[
            pl.BlockSpec((BN, H), lambda i: (i, 0)),
            pl.BlockSpec((BN, H // 2), lambda i: (i, 0)),
        ],
        out_shape=[
            jax.ShapeDtypeStruct((n_nodes, H), jnp.float32),
            jax.ShapeDtypeStruct((n_nodes, H // 2), jnp.int32),
        ],
    )(x, a0, a1, W1a, b1a.reshape(1, -1), W1b, b1b.reshape(1, -1))


def _mlp2_body(G, h, a0, a1, w2a, b2a, w2b, b2b, wr1, br1, wr2, br2, batch,
               pi_out, pooled_out, acc):
    m = h[...] + a0[...] + a1[...]
    t = _relu(jnp.dot(m, w2a[...], preferred_element_type=jnp.float32) + b2a[...])
    t = _relu(jnp.dot(t, w2b[...], preferred_element_type=jnp.float32) + b2b[...])
    r = _relu(jnp.dot(t, wr1[...], preferred_element_type=jnp.float32) + br1[...])
    z = jnp.dot(r, wr2[...], preferred_element_type=jnp.float32) + br2[...]
    pi = jax.nn.sigmoid(z[:, 0])
    pi_out[0, 0, :] = pi
    b = batch[0, 0, :]
    onehot = (b[:, None] == lax.broadcasted_iota(jnp.int32, (b.shape[0], G), 1))
    contrib = jnp.sum(jnp.where(onehot, pi[:, None], 0.0), axis=0)

    @pl.when(pl.program_id(0) == 0)
    def _():
        acc[...] = jnp.zeros_like(acc)

    acc[...] += contrib[None, :]
    pooled_out[...] = acc[...]


def _node_mlp2(h, a0, a1, W2a, b2a, W2b, b2b, Wr1, br1, Wr2, br2, batch3, G):
    n_nodes, H = h.shape
    HR = Wr1.shape[1]
    nb = n_nodes // BN
    return pl.pallas_call(
        functools.partial(_mlp2_body, G),
        grid=(nb,),
        in_specs=[
            pl.BlockSpec((BN, H), lambda i: (i, 0)),
            pl.BlockSpec((BN, H), lambda i: (i, 0)),
            pl.BlockSpec((BN, H), lambda i: (i, 0)),
            pl.BlockSpec((H, H), lambda i: (0, 0)),
            pl.BlockSpec((1, H), lambda i: (0, 0)),
            pl.BlockSpec((H, H), lambda i: (0, 0)),
            pl.BlockSpec((1, H), lambda i: (0, 0)),
            pl.BlockSpec((H, HR), lambda i: (0, 0)),
            pl.BlockSpec((1, HR), lambda i: (0, 0)),
            pl.BlockSpec((HR, 1), lambda i: (0, 0)),
            pl.BlockSpec((1, 1), lambda i: (0, 0)),
            pl.BlockSpec((1, 1, BN), lambda i: (i, 0, 0)),
        ],
        out_specs=[
            pl.BlockSpec((1, 1, BN), lambda i: (i, 0, 0)),
            pl.BlockSpec((1, G), lambda i: (0, 0)),
        ],
        out_shape=[
            jax.ShapeDtypeStruct((nb, 1, BN), jnp.float32),
            jax.ShapeDtypeStruct((1, G), jnp.float32),
        ],
        scratch_shapes=[pltpu.VMEM((1, G), jnp.float32)],
    )(h, a0, a1, W2a, b2a.reshape(1, -1), W2b, b2b.reshape(1, -1),
      Wr1, br1.reshape(1, -1), Wr2, br2.reshape(1, -1), batch3)


def _final_body(G, pi, batch, btot, pooled, out):
    p = pi[0, 0, :]
    b = batch[0, 0, :]
    onehot = (b[:, None] == lax.broadcasted_iota(jnp.int32, (b.shape[0], G), 1))
    B_b = jnp.sum(jnp.where(onehot, btot[...], 0.0), axis=1)
    exp_b = jnp.sum(jnp.where(onehot, pooled[...], 0.0), axis=1)
    ratio = jnp.minimum(B_b / (exp_b + 1e-12), 1.0)
    out[0, 0, :] = p * ratio


def _finalize(pi3, batch3, B_total, pooled, G):
    nb = pi3.shape[0]
    return pl.pallas_call(
        functools.partial(_final_body, G),
        grid=(nb,),
        in_specs=[
            pl.BlockSpec((1, 1, BN), lambda i: (i, 0, 0)),
            pl.BlockSpec((1, 1, BN), lambda i: (i, 0, 0)),
            pl.BlockSpec((1, G), lambda i: (0, 0)),
            pl.BlockSpec((1, G), lambda i: (0, 0)),
        ],
        out_specs=pl.BlockSpec((1, 1, BN), lambda i: (i, 0, 0)),
        out_shape=jax.ShapeDtypeStruct((nb, 1, BN), jnp.float32),
    )(pi3, batch3, B_total.reshape(1, G), pooled)


def kernel(x, edge_index, edge_attr, batch, B_total,
           We1, be1, W1a, b1a, W1b, b1b,
           We2, be2, W2a, b2a, W2b, b2b,
           Wr1, br1, Wr2, br2):
    n_nodes, DF = x.shape
    E = edge_index.shape[1]
    DE = edge_attr.shape[1]
    H = W1a.shape[1]
    G = B_total.shape[0]

    CHB_BASE = 128
    CPW = -(-E // (NW * CHB_BASE))
    CPW += CPW % 2
    EP = NW * CPW * CHB_BASE
    NP = ((n_nodes + 1 + 127) // 128) * 128

    pad = EP - E
    src = jnp.concatenate([edge_index[0], jnp.zeros((pad,), jnp.int32)])
    dst = jnp.concatenate([edge_index[1],
                           jnp.full((pad,), n_nodes, jnp.int32)])
    ea_pad = jnp.concatenate(
        [edge_attr, jnp.zeros((pad, DE), jnp.float32)], axis=0)

    CH1, CH2 = 64, 128
    CPW1, CPW2 = EP // (NW * CH1), EP // (NW * CH2)
    srcw1 = src.reshape(NW, CPW1, CH1)
    dstw1 = dst.reshape(NW, CPW1, CH1)
    srcw2 = src.reshape(NW, CPW2, CH2)
    dstw2 = dst.reshape(NW, CPW2, CH2)

    ef1p = _edge_lin(ea_pad, We1, be1, BE=NW * CHB_BASE)
    ef2p = _edge_lin(ea_pad, We2, be2, BE=NW * CHB_BASE)
    ef1w = ef1p.reshape(NW, CPW1, CH1, DF // 2)
    ef2w = ef2p.reshape(NW, CPW2, CH2, H // 2)

    z128 = jnp.zeros((NP // NS, DF), jnp.float32)
    z64 = jnp.zeros((NP // NS, H), jnp.float32)

    xp = _pack_rows(x)
    agg1 = _sc_conv(xp, srcw1, dstw1, ef1w, z128, NP, DF)
    h1, h1p = _node_mlp1(x, agg1[0, :n_nodes], agg1[1, :n_nodes],
                         W1a, b1a, W1b, b1b)

    agg2 = _sc_conv(h1p, srcw2, dstw2, ef2w, z64, NP, H)

    batch3 = batch.reshape(-1, 1, BN)
    pi3, pooled = _node_mlp2(h1, agg2[0, :n_nodes], agg2[1, :n_nodes],
                             W2a, b2a, W2b, b2b, Wr1, br1, Wr2, br2,
                             batch3, G)
    out3 = _finalize(pi3, batch3, B_total, pooled, G)
    return out3.reshape(n_nodes)


# speedup vs baseline: 3.0858x; 1.0195x over previous
import functools

import jax
import jax.numpy as jnp
from jax import lax
from jax.experimental import pallas as pl
from jax.experimental.pallas import tpu as pltpu
from jax.experimental.pallas import tpu_sc as plsc

NC = 2
NS = 16
NW = NC * NS
LANES = 16
BN = 1000
MASK_HI = jnp.int32(-65536)


def _pack_bf16_pair(lo, hi):
    bl = lax.bitcast_convert_type(lo, jnp.uint32)
    bh = lax.bitcast_convert_type(hi, jnp.uint32)
    bl = (bl + 0x7FFF + ((bl >> 16) & 1)) >> 16
    bh = (bh + 0x7FFF + ((bh >> 16) & 1)) >> 16
    return lax.bitcast_convert_type(bl | (bh << 16), jnp.int32)


def _edge_lin_body(ea, we, be, efp):
    ef = jnp.dot(ea[...], we[...], preferred_element_type=jnp.float32) + be[...]
    d2 = ef.shape[1] // 2
    efp[...] = _pack_bf16_pair(ef[:, :d2], ef[:, d2:])


def _edge_lin(ea_pad, We, be, BE):
    EP, DE = ea_pad.shape
    D = We.shape[1]
    return pl.pallas_call(
        _edge_lin_body,
        grid=(EP // BE,),
        in_specs=[
            pl.BlockSpec((BE, DE), lambda i: (i, 0)),
            pl.BlockSpec((DE, D), lambda i: (0, 0)),
            pl.BlockSpec((1, D), lambda i: (0, 0)),
        ],
        out_specs=pl.BlockSpec((BE, D // 2), lambda i: (i, 0)),
        out_shape=jax.ShapeDtypeStruct((EP, D // 2), jnp.int32),
    )(ea_pad, We, be.reshape(1, -1))


def _pack_rows_body(x, xp):
    d2 = x.shape[1] // 2
    xp[...] = _pack_bf16_pair(x[..., :d2], x[..., d2:])


def _pack_rows(x):
    n, D = x.shape
    return pl.pallas_call(
        _pack_rows_body,
        grid=(n // BN,),
        in_specs=[pl.BlockSpec((BN, D), lambda i: (i, 0))],
        out_specs=pl.BlockSpec((BN, D // 2), lambda i: (i, 0)),
        out_shape=jax.ShapeDtypeStruct((n, D // 2), jnp.int32),
    )(x)


def _sc_conv(xp, srcw, dstw, efw, zrows, NP, D):
    D2 = D // 2
    CPW, CHB = srcw.shape[1], srcw.shape[2]
    stripe = NP // NS
    mesh = plsc.VectorSubcoreMesh(core_axis_name="c", subcore_axis_name="s",
                                  num_cores=NC, num_subcores=NS)

    @functools.partial(
        pl.kernel,
        out_type=jax.ShapeDtypeStruct((NC, NP, D), jnp.float32),
        mesh=mesh,
        compiler_params=pltpu.CompilerParams(use_tc_tiling_on_sc=False,
                                             needs_layout_passes=False),
        scratch_types=[
            pltpu.VMEM((CPW, CHB), jnp.int32),
            pltpu.VMEM((CPW, CHB), jnp.int32),
            pltpu.VMEM((CHB, D2), jnp.int32),
            pltpu.VMEM((CHB, D2), jnp.int32),
            pltpu.VMEM((CHB, D2), jnp.int32),
            pltpu.VMEM((CHB, D2), jnp.int32),
            pltpu.VMEM((CHB, D), jnp.float32),
            pltpu.VMEM_SHARED((NP, D), jnp.float32),
            pltpu.SemaphoreType.DMA,
            pltpu.SemaphoreType.DMA,
            pltpu.SemaphoreType.DMA,
            pltpu.SemaphoreType.DMA,
        ],
    )
    def conv(x_hbm, src_hbm, dst_hbm, ef_hbm, z_hbm, out_hbm,
             src_all, dst_all, rows0, rows1, ef0, ef1, msg, acc,
             semr0, semr1, seme0, seme1):
        c = lax.axis_index("c")
        s = lax.axis_index("s")
        wid = c * NS + s
        zvec = jnp.zeros((LANES,), jnp.float32)
        rows = (rows0, rows1)
        efs = (ef0, ef1)
        semr = (semr0, semr1)
        seme = (seme0, seme1)

        pltpu.sync_copy(src_hbm.at[wid], src_all)
        pltpu.sync_copy(dst_hbm.at[wid], dst_all)
        pltpu.sync_copy(z_hbm, acc.at[pl.ds(s * stripe, stripe)])
        plsc.subcore_barrier()

        def issue(j, b):
            pltpu.async_copy(x_hbm.at[src_all.at[j]], rows[b], semr[b])
            pltpu.async_copy(ef_hbm.at[wid, j], efs[b], seme[b])

        def wait(j, b):
            pltpu.make_async_copy(x_hbm.at[src_all.at[j]], rows[b], semr[b]).wait()
            pltpu.make_async_copy(ef_hbm.at[wid, j], efs[b], seme[b]).wait()

        zb = jnp.zeros((2 * LANES,), jnp.bfloat16)

        def process(j, b):
            rv, ev = rows[b], efs[b]

            @pl.loop(0, CHB, unroll=4)
            def _edge(r):
                for k in range(D2 // LANES):
                    sl = pl.ds(k * LANES, LANES)
                    m = jnp.maximum(plsc.bitcast(rv[r, sl], jnp.bfloat16)
                                    + plsc.bitcast(ev[r, sl], jnp.bfloat16), zb)
                    m32 = plsc.bitcast(m, jnp.int32)
                    msg[r, sl] = plsc.bitcast(m32 << 16, jnp.float32)
                    msg[r, pl.ds(D2 + k * LANES, LANES)] = plsc.bitcast(
                        m32 & MASK_HI, jnp.float32)

            pltpu.sync_copy(msg, acc.at[dst_all.at[j]], add=True)

        issue(0, 0)

        @pl.loop(0, CPW // 2)
        def _pair(p):
            j0 = 2 * p
            issue(j0 + 1, 1)
            wait(j0, 0)
            process(j0, 0)

            @pl.when(p + 1 < CPW // 2)
            def _():
                issue(j0 + 2, 0)

            wait(j0 + 1, 1)
            process(j0 + 1, 1)

        plsc.subcore_barrier()
        pltpu.sync_copy(acc.at[pl.ds(s * stripe, stripe)],
                        out_hbm.at[c, pl.ds(s * stripe, stripe)])

    return conv(xp, srcw, dstw, efw, zrows)


def _relu(v):
    return jnp.maximum(v, 0.0)


def _mlp1_body(x, a0, a1, wa, ba, wb, bb, out, outp):
    m = x[...] + a0[0] + a1[0]
    t = _relu(jnp.dot(m, wa[...], preferred_element_type=jnp.float32) + ba[...])
    h = _relu(jnp.dot(t, wb[...], preferred_element_type=jnp.float32) + bb[...])
    out[...] = h
    d2 = h.shape[1] // 2
    outp[...] = _pack_bf16_pair(h[:, :d2], h[:, d2:])


def _node_mlp1(x, agg, W1a, b1a, W1b, b1b):
    n_nodes, DF = x.shape
    H = W1a.shape[1]
    return pl.pallas_call(
        _mlp1_body,
        grid=(n_nodes // BN,),
        in_specs=[
            pl.BlockSpec((BN, DF), lambda i: (i, 0)),
            pl.BlockSpec((1, BN, DF), lambda i: (0, i, 0)),
            pl.BlockSpec((1, BN, DF), lambda i: (1, i, 0)),
            pl.BlockSpec((DF, H), lambda i: (0, 0)),
            pl.BlockSpec((1, H), lambda i: (0, 0)),
            pl.BlockSpec((H, H), lambda i: (0, 0)),
            pl.BlockSpec((1, H), lambda i: (0, 0)),
        ],
        out_specs=[
            pl.BlockSpec((BN, H), lambda i: (i, 0)),
            pl.BlockSpec((BN, H // 2), lambda i: (i, 0)),
        ],
        out_shape=[
            jax.ShapeDtypeStruct((n_nodes, H), jnp.float32),
            jax.ShapeDtypeStruct((n_nodes, H // 2), jnp.int32),
        ],
    )(x, agg, agg, W1a, b1a.reshape(1, -1), W1b, b1b.reshape(1, -1))


def _mlp2_body(G, h, a0, a1, w2a, b2a, w2b, b2b, wr1, br1, wr2, br2, batch,
               pi_out, pooled_out, acc):
    m = h[...] + a0[0] + a1[0]
    t = _relu(jnp.dot(m, w2a[...], preferred_element_type=jnp.float32) + b2a[...])
    t = _relu(jnp.dot(t, w2b[...], preferred_element_type=jnp.float32) + b2b[...])
    r = _relu(jnp.dot(t, wr1[...], preferred_element_type=jnp.float32) + br1[...])
    z = jnp.dot(r, wr2[...], preferred_element_type=jnp.float32) + br2[...]
    pi = jax.nn.sigmoid(z[:, 0])
    pi_out[0, 0, :] = pi
    b = batch[0, 0, :]
    onehot = (b[:, None] == lax.broadcasted_iota(jnp.int32, (b.shape[0], G), 1))
    contrib = jnp.sum(jnp.where(onehot, pi[:, None], 0.0), axis=0)

    @pl.when(pl.program_id(0) == 0)
    def _():
        acc[...] = jnp.zeros_like(acc)

    acc[...] += contrib[None, :]
    pooled_out[...] = acc[...]


def _node_mlp2(h, agg, W2a, b2a, W2b, b2b, Wr1, br1, Wr2, br2, batch3, G):
    n_nodes, H = h.shape
    HR = Wr1.shape[1]
    nb = n_nodes // BN
    return pl.pallas_call(
        functools.partial(_mlp2_body, G),
        grid=(nb,),
        in_specs=[
            pl.BlockSpec((BN, H), lambda i: (i, 0)),
            pl.BlockSpec((1, BN, H), lambda i: (0, i, 0)),
            pl.BlockSpec((1, BN, H), lambda i: (1, i, 0)),
            pl.BlockSpec((H, H), lambda i: (0, 0)),
            pl.BlockSpec((1, H), lambda i: (0, 0)),
            pl.BlockSpec((H, H), lambda i: (0, 0)),
            pl.BlockSpec((1, H), lambda i: (0, 0)),
            pl.BlockSpec((H, HR), lambda i: (0, 0)),
            pl.BlockSpec((1, HR), lambda i: (0, 0)),
            pl.BlockSpec((HR, 1), lambda i: (0, 0)),
            pl.BlockSpec((1, 1), lambda i: (0, 0)),
            pl.BlockSpec((1, 1, BN), lambda i: (i, 0, 0)),
        ],
        out_specs=[
            pl.BlockSpec((1, 1, BN), lambda i: (i, 0, 0)),
            pl.BlockSpec((1, G), lambda i: (0, 0)),
        ],
        out_shape=[
            jax.ShapeDtypeStruct((nb, 1, BN), jnp.float32),
            jax.ShapeDtypeStruct((1, G), jnp.float32),
        ],
        scratch_shapes=[pltpu.VMEM((1, G), jnp.float32)],
    )(h, agg, agg, W2a, b2a.reshape(1, -1), W2b, b2b.reshape(1, -1),
      Wr1, br1.reshape(1, -1), Wr2, br2.reshape(1, -1), batch3)


def _final_body(G, pi, batch, btot, pooled, out):
    p = pi[0, 0, :]
    b = batch[0, 0, :]
    onehot = (b[:, None] == lax.broadcasted_iota(jnp.int32, (b.shape[0], G), 1))
    B_b = jnp.sum(jnp.where(onehot, btot[...], 0.0), axis=1)
    exp_b = jnp.sum(jnp.where(onehot, pooled[...], 0.0), axis=1)
    ratio = jnp.minimum(B_b / (exp_b + 1e-12), 1.0)
    out[0, 0, :] = p * ratio


def _finalize(pi3, batch3, B_total, pooled, G):
    nb = pi3.shape[0]
    return pl.pallas_call(
        functools.partial(_final_body, G),
        grid=(nb,),
        in_specs=[
            pl.BlockSpec((1, 1, BN), lambda i: (i, 0, 0)),
            pl.BlockSpec((1, 1, BN), lambda i: (i, 0, 0)),
            pl.BlockSpec((1, G), lambda i: (0, 0)),
            pl.BlockSpec((1, G), lambda i: (0, 0)),
        ],
        out_specs=pl.BlockSpec((1, 1, BN), lambda i: (i, 0, 0)),
        out_shape=jax.ShapeDtypeStruct((nb, 1, BN), jnp.float32),
    )(pi3, batch3, B_total.reshape(1, G), pooled)


def kernel(x, edge_index, edge_attr, batch, B_total,
           We1, be1, W1a, b1a, W1b, b1b,
           We2, be2, W2a, b2a, W2b, b2b,
           Wr1, br1, Wr2, br2):
    n_nodes, DF = x.shape
    E = edge_index.shape[1]
    DE = edge_attr.shape[1]
    H = W1a.shape[1]
    G = B_total.shape[0]

    CHB_BASE = 128
    CPW = -(-E // (NW * CHB_BASE))
    CPW += CPW % 2
    EP = NW * CPW * CHB_BASE
    NP = ((n_nodes + 1 + 127) // 128) * 128

    pad = EP - E
    src = jnp.concatenate([edge_index[0], jnp.zeros((pad,), jnp.int32)])
    dst = jnp.concatenate([edge_index[1],
                           jnp.full((pad,), n_nodes, jnp.int32)])
    ea_pad = jnp.concatenate(
        [edge_attr, jnp.zeros((pad, DE), jnp.float32)], axis=0)

    CH1, CH2 = 64, 128
    CPW1, CPW2 = EP // (NW * CH1), EP // (NW * CH2)
    srcw1 = src.reshape(NW, CPW1, CH1)
    dstw1 = dst.reshape(NW, CPW1, CH1)
    srcw2 = src.reshape(NW, CPW2, CH2)
    dstw2 = dst.reshape(NW, CPW2, CH2)

    ef1p = _edge_lin(ea_pad, We1, be1, BE=NW * CHB_BASE)
    ef2p = _edge_lin(ea_pad, We2, be2, BE=NW * CHB_BASE)
    ef1w = ef1p.reshape(NW, CPW1, CH1, DF // 2)
    ef2w = ef2p.reshape(NW, CPW2, CH2, H // 2)

    z128 = jnp.zeros((NP // NS, DF), jnp.float32)
    z64 = jnp.zeros((NP // NS, H), jnp.float32)

    xp = _pack_rows(x)
    agg1 = _sc_conv(xp, srcw1, dstw1, ef1w, z128, NP, DF)
    h1, h1p = _node_mlp1(x, agg1, W1a, b1a, W1b, b1b)

    agg2 = _sc_conv(h1p, srcw2, dstw2, ef2w, z64, NP, H)

    batch3 = batch.reshape(-1, 1, BN)
    pi3, pooled = _node_mlp2(h1, agg2, W2a, b2a, W2b, b2b, Wr1, br1, Wr2, br2,
                             batch3, G)
    out3 = _finalize(pi3, batch3, B_total, pooled, G)
    return out3.reshape(n_nodes)


# speedup vs baseline: 3.1748x; 1.0289x over previous
import functools

import jax
import jax.numpy as jnp
from jax import lax
from jax.experimental import pallas as pl
from jax.experimental.pallas import tpu as pltpu
from jax.experimental.pallas import tpu_sc as plsc

NC = 2
NS = 16
NW = NC * NS
LANES = 16
BN = 1000
MASK_HI = jnp.int32(-65536)


def _pack_bf16_pair(lo, hi):
    bl = lax.bitcast_convert_type(lo, jnp.uint32)
    bh = lax.bitcast_convert_type(hi, jnp.uint32)
    bl = (bl + 0x7FFF + ((bl >> 16) & 1)) >> 16
    bh = (bh + 0x7FFF + ((bh >> 16) & 1)) >> 16
    return lax.bitcast_convert_type(bl | (bh << 16), jnp.int32)


def _edge_lin_body(ea, we, be, efp):
    ef = jnp.dot(ea[...], we[...], preferred_element_type=jnp.float32) + be[...]
    d2 = ef.shape[1] // 2
    efp[...] = _pack_bf16_pair(ef[:, :d2], ef[:, d2:])


def _edge_lin(ea_pad, We, be, BE):
    EP, DE = ea_pad.shape
    D = We.shape[1]
    return pl.pallas_call(
        _edge_lin_body,
        grid=(EP // BE,),
        in_specs=[
            pl.BlockSpec((BE, DE), lambda i: (i, 0)),
            pl.BlockSpec((DE, D), lambda i: (0, 0)),
            pl.BlockSpec((1, D), lambda i: (0, 0)),
        ],
        out_specs=pl.BlockSpec((BE, D // 2), lambda i: (i, 0)),
        out_shape=jax.ShapeDtypeStruct((EP, D // 2), jnp.int32),
    )(ea_pad, We, be.reshape(1, -1))


def _pack_rows_body(x, xp):
    d2 = x.shape[1] // 2
    xp[...] = _pack_bf16_pair(x[..., :d2], x[..., d2:])


def _pack_rows(x):
    n, D = x.shape
    return pl.pallas_call(
        _pack_rows_body,
        grid=(n // BN,),
        in_specs=[pl.BlockSpec((BN, D), lambda i: (i, 0))],
        out_specs=pl.BlockSpec((BN, D // 2), lambda i: (i, 0)),
        out_shape=jax.ShapeDtypeStruct((n, D // 2), jnp.int32),
    )(x)


def _sc_conv(xp, srcw, dstw, efw, zrows, NP, D):
    D2 = D // 2
    CPW, CHB = srcw.shape[1], srcw.shape[2]
    stripe = NP // NS
    mesh = plsc.VectorSubcoreMesh(core_axis_name="c", subcore_axis_name="s",
                                  num_cores=NC, num_subcores=NS)

    @functools.partial(
        pl.kernel,
        out_type=jax.ShapeDtypeStruct((NC, NP, D), jnp.float32),
        mesh=mesh,
        compiler_params=pltpu.CompilerParams(use_tc_tiling_on_sc=False,
                                             needs_layout_passes=False),
        scratch_types=[
            pltpu.VMEM((CPW, CHB), jnp.int32),
            pltpu.VMEM((CHB,), jnp.int32),
            pltpu.VMEM((CHB,), jnp.int32),
            pltpu.VMEM((CHB,), jnp.int32),
            pltpu.VMEM((CHB,), jnp.int32),
            pltpu.VMEM((CHB, D2), jnp.int32),
            pltpu.VMEM((CHB, D2), jnp.int32),
            pltpu.VMEM((CHB, D2), jnp.int32),
            pltpu.VMEM((CHB, D2), jnp.int32),
            pltpu.VMEM((CHB, D), jnp.float32),
            pltpu.VMEM((CHB, D), jnp.float32),
            pltpu.VMEM_SHARED((NP, D), jnp.float32),
            pltpu.SemaphoreType.DMA,
            pltpu.SemaphoreType.DMA,
            pltpu.SemaphoreType.DMA,
            pltpu.SemaphoreType.DMA,
            pltpu.SemaphoreType.DMA,
            pltpu.SemaphoreType.DMA,
            pltpu.SemaphoreType.DMA,
            pltpu.SemaphoreType.DMA,
            pltpu.SemaphoreType.DMA,
            pltpu.SemaphoreType.DMA,
        ],
    )
    def conv(x_hbm, src_hbm, dst_hbm, ef_hbm, z_hbm, out_hbm,
             src_all, dst0, dst1, dst2, dst3, rows0, rows1, ef0, ef1,
             msg0, msg1, acc,
             semr0, semr1, seme0, seme1,
             semd0, semd1, semd2, semd3, sems0, sems1):
        c = lax.axis_index("c")
        s = lax.axis_index("s")
        wid = c * NS + s
        rows = (rows0, rows1)
        efs = (ef0, ef1)
        dsts = (dst0, dst1, dst2, dst3)
        msgs = (msg0, msg1)
        semr = (semr0, semr1)
        seme = (seme0, seme1)
        semd = (semd0, semd1, semd2, semd3)
        sems = (sems0, sems1)

        pltpu.sync_copy(src_hbm.at[wid], src_all)
        pltpu.sync_copy(z_hbm, acc.at[pl.ds(s * stripe, stripe)])
        plsc.subcore_barrier()

        def issue(j, b, d):
            pltpu.async_copy(x_hbm.at[src_all.at[j]], rows[b], semr[b])
            pltpu.async_copy(ef_hbm.at[wid, j], efs[b], seme[b])
            pltpu.async_copy(dst_hbm.at[wid, j], dsts[d], semd[d])

        def wait(j, b, d):
            pltpu.make_async_copy(x_hbm.at[src_all.at[j]], rows[b], semr[b]).wait()
            pltpu.make_async_copy(ef_hbm.at[wid, j], efs[b], seme[b]).wait()
            pltpu.make_async_copy(dst_hbm.at[wid, j], dsts[d], semd[d]).wait()

        def wait_scatter(b, d):
            pltpu.make_async_copy(msgs[b], acc.at[dsts[d]], sems[b]).wait()

        zb = jnp.zeros((2 * LANES,), jnp.bfloat16)

        def compute_and_scatter(b, d):
            rv, ev, mg = rows[b], efs[b], msgs[b]

            @pl.loop(0, CHB, unroll=4)
            def _edge(r):
                for k in range(D2 // LANES):
                    sl = pl.ds(k * LANES, LANES)
                    m = jnp.maximum(plsc.bitcast(rv[r, sl], jnp.bfloat16)
                                    + plsc.bitcast(ev[r, sl], jnp.bfloat16), zb)
                    m32 = plsc.bitcast(m, jnp.int32)
                    mg[r, sl] = plsc.bitcast(m32 << 16, jnp.float32)
                    mg[r, pl.ds(D2 + k * LANES, LANES)] = plsc.bitcast(
                        m32 & MASK_HI, jnp.float32)

            pltpu.async_copy(mg, acc.at[dsts[d]], sems[b], add=True)

        issue(0, 0, 0)
        NQ = CPW // 4

        @pl.loop(0, NQ)
        def _quad(q):
            for t in range(4):
                j = 4 * q + t
                bb, dd = t % 2, t
                nb, nd = (t + 1) % 2, (t + 1) % 4

                @pl.when(j + 1 < CPW)
                def _():
                    issue(j + 1, nb, nd)

                wait(j, bb, dd)
                if t >= 2:
                    wait_scatter(bb, t - 2)
                else:
                    @pl.when(q >= 1)
                    def _():
                        wait_scatter(bb, t + 2)

                compute_and_scatter(bb, dd)

        wait_scatter(0, 2)
        wait_scatter(1, 3)
        plsc.subcore_barrier()
        pltpu.sync_copy(acc.at[pl.ds(s * stripe, stripe)],
                        out_hbm.at[c, pl.ds(s * stripe, stripe)])

    return conv(xp, srcw, dstw, efw, zrows)


def _relu(v):
    return jnp.maximum(v, 0.0)


def _mlp1_body(x, a0, a1, wa, ba, wb, bb, out, outp):
    m = x[...] + a0[0] + a1[0]
    t = _relu(jnp.dot(m, wa[...], preferred_element_type=jnp.float32) + ba[...])
    h = _relu(jnp.dot(t, wb[...], preferred_element_type=jnp.float32) + bb[...])
    out[...] = h
    d2 = h.shape[1] // 2
    outp[...] = _pack_bf16_pair(h[:, :d2], h[:, d2:])


def _node_mlp1(x, agg, W1a, b1a, W1b, b1b):
    n_nodes, DF = x.shape
    H = W1a.shape[1]
    return pl.pallas_call(
        _mlp1_body,
        grid=(n_nodes // BN,),
        in_specs=[
            pl.BlockSpec((BN, DF), lambda i: (i, 0)),
            pl.BlockSpec((1, BN, DF), lambda i: (0, i, 0)),
            pl.BlockSpec((1, BN, DF), lambda i: (1, i, 0)),
            pl.BlockSpec((DF, H), lambda i: (0, 0)),
            pl.BlockSpec((1, H), lambda i: (0, 0)),
            pl.BlockSpec((H, H), lambda i: (0, 0)),
            pl.BlockSpec((1, H), lambda i: (0, 0)),
        ],
        out_specs=[
            pl.BlockSpec((BN, H), lambda i: (i, 0)),
            pl.BlockSpec((BN, H // 2), lambda i: (i, 0)),
        ],
        out_shape=[
            jax.ShapeDtypeStruct((n_nodes, H), jnp.float32),
            jax.ShapeDtypeStruct((n_nodes, H // 2), jnp.int32),
        ],
    )(x, agg, agg, W1a, b1a.reshape(1, -1), W1b, b1b.reshape(1, -1))


def _mlp2_body(G, h, a0, a1, w2a, b2a, w2b, b2b, wr1, br1, wr2, br2, batch,
               pi_out, pooled_out, acc):
    m = h[...] + a0[0] + a1[0]
    t = _relu(jnp.dot(m, w2a[...], preferred_element_type=jnp.float32) + b2a[...])
    t = _relu(jnp.dot(t, w2b[...], preferred_element_type=jnp.float32) + b2b[...])
    r = _relu(jnp.dot(t, wr1[...], preferred_element_type=jnp.float32) + br1[...])
    z = jnp.dot(r, wr2[...], preferred_element_type=jnp.float32) + br2[...]
    pi = jax.nn.sigmoid(z[:, 0])
    pi_out[0, 0, :] = pi
    b = batch[0, 0, :]
    onehot = (b[:, None] == lax.broadcasted_iota(jnp.int32, (b.shape[0], G), 1))
    contrib = jnp.sum(jnp.where(onehot, pi[:, None], 0.0), axis=0)

    @pl.when(pl.program_id(0) == 0)
    def _():
        acc[...] = jnp.zeros_like(acc)

    acc[...] += contrib[None, :]
    pooled_out[...] = acc[...]


def _node_mlp2(h, agg, W2a, b2a, W2b, b2b, Wr1, br1, Wr2, br2, batch3, G):
    n_nodes, H = h.shape
    HR = Wr1.shape[1]
    nb = n_nodes // BN
    return pl.pallas_call(
        functools.partial(_mlp2_body, G),
        grid=(nb,),
        in_specs=[
            pl.BlockSpec((BN, H), lambda i: (i, 0)),
            pl.BlockSpec((1, BN, H), lambda i: (0, i, 0)),
            pl.BlockSpec((1, BN, H), lambda i: (1, i, 0)),
            pl.BlockSpec((H, H), lambda i: (0, 0)),
            pl.BlockSpec((1, H), lambda i: (0, 0)),
            pl.BlockSpec((H, H), lambda i: (0, 0)),
            pl.BlockSpec((1, H), lambda i: (0, 0)),
            pl.BlockSpec((H, HR), lambda i: (0, 0)),
            pl.BlockSpec((1, HR), lambda i: (0, 0)),
            pl.BlockSpec((HR, 1), lambda i: (0, 0)),
            pl.BlockSpec((1, 1), lambda i: (0, 0)),
            pl.BlockSpec((1, 1, BN), lambda i: (i, 0, 0)),
        ],
        out_specs=[
            pl.BlockSpec((1, 1, BN), lambda i: (i, 0, 0)),
            pl.BlockSpec((1, G), lambda i: (0, 0)),
        ],
        out_shape=[
            jax.ShapeDtypeStruct((nb, 1, BN), jnp.float32),
            jax.ShapeDtypeStruct((1, G), jnp.float32),
        ],
        scratch_shapes=[pltpu.VMEM((1, G), jnp.float32)],
    )(h, agg, agg, W2a, b2a.reshape(1, -1), W2b, b2b.reshape(1, -1),
      Wr1, br1.reshape(1, -1), Wr2, br2.reshape(1, -1), batch3)


def _final_body(G, pi, batch, btot, pooled, out):
    p = pi[0, 0, :]
    b = batch[0, 0, :]
    onehot = (b[:, None] == lax.broadcasted_iota(jnp.int32, (b.shape[0], G), 1))
    B_b = jnp.sum(jnp.where(onehot, btot[...], 0.0), axis=1)
    exp_b = jnp.sum(jnp.where(onehot, pooled[...], 0.0), axis=1)
    ratio = jnp.minimum(B_b / (exp_b + 1e-12), 1.0)
    out[0, 0, :] = p * ratio


def _finalize(pi3, batch3, B_total, pooled, G):
    nb = pi3.shape[0]
    return pl.pallas_call(
        functools.partial(_final_body, G),
        grid=(nb,),
        in_specs=[
            pl.BlockSpec((1, 1, BN), lambda i: (i, 0, 0)),
            pl.BlockSpec((1, 1, BN), lambda i: (i, 0, 0)),
            pl.BlockSpec((1, G), lambda i: (0, 0)),
            pl.BlockSpec((1, G), lambda i: (0, 0)),
        ],
        out_specs=pl.BlockSpec((1, 1, BN), lambda i: (i, 0, 0)),
        out_shape=jax.ShapeDtypeStruct((nb, 1, BN), jnp.float32),
    )(pi3, batch3, B_total.reshape(1, G), pooled)


def kernel(x, edge_index, edge_attr, batch, B_total,
           We1, be1, W1a, b1a, W1b, b1b,
           We2, be2, W2a, b2a, W2b, b2b,
           Wr1, br1, Wr2, br2):
    n_nodes, DF = x.shape
    E = edge_index.shape[1]
    DE = edge_attr.shape[1]
    H = W1a.shape[1]
    G = B_total.shape[0]

    CHB_BASE = 128
    CPW = -(-E // (NW * CHB_BASE))
    CPW += CPW % 2
    EP = NW * CPW * CHB_BASE
    NP = ((n_nodes + 1 + 127) // 128) * 128

    pad = EP - E
    src = jnp.concatenate([edge_index[0], jnp.zeros((pad,), jnp.int32)])
    dst = jnp.concatenate([edge_index[1],
                           jnp.full((pad,), n_nodes, jnp.int32)])
    ea_pad = jnp.concatenate(
        [edge_attr, jnp.zeros((pad, DE), jnp.float32)], axis=0)

    CH1, CH2 = 64, 128
    CPW1, CPW2 = EP // (NW * CH1), EP // (NW * CH2)
    srcw1 = src.reshape(NW, CPW1, CH1)
    dstw1 = dst.reshape(NW, CPW1, CH1)
    srcw2 = src.reshape(NW, CPW2, CH2)
    dstw2 = dst.reshape(NW, CPW2, CH2)

    ef1p = _edge_lin(ea_pad, We1, be1, BE=NW * CHB_BASE)
    ef2p = _edge_lin(ea_pad, We2, be2, BE=NW * CHB_BASE)
    ef1w = ef1p.reshape(NW, CPW1, CH1, DF // 2)
    ef2w = ef2p.reshape(NW, CPW2, CH2, H // 2)

    z128 = jnp.zeros((NP // NS, DF), jnp.float32)
    z64 = jnp.zeros((NP // NS, H), jnp.float32)

    xp = _pack_rows(x)
    agg1 = _sc_conv(xp, srcw1, dstw1, ef1w, z128, NP, DF)
    h1, h1p = _node_mlp1(x, agg1, W1a, b1a, W1b, b1b)

    agg2 = _sc_conv(h1p, srcw2, dstw2, ef2w, z64, NP, H)

    batch3 = batch.reshape(-1, 1, BN)
    pi3, pooled = _node_mlp2(h1, agg2, W2a, b2a, W2b, b2b, Wr1, br1, Wr2, br2,
                             batch3, G)
    out3 = _finalize(pi3, batch3, B_total, pooled, G)
    return out3.reshape(n_nodes)
